# baseline (speedup 1.0000x reference)
import functools

import jax
import jax.numpy as jnp
from jax import lax
from jax.experimental import pallas as pl
from jax.experimental.pallas import tpu as pltpu
from jax.experimental.pallas import tpu_sc as plsc

D = 1024
HID = 2048
DK = 64
DV = 64
H = 16
FSEG = 2048
SEG = 256
B = 2
S = 8192
NROW = B * (S // FSEG)
NSEL = S // 8
TSEL = B * NSEL


def _scores_body(x_ref, w_ref, b_ref, o_ref):
    xb = x_ref[0]
    res = jnp.dot(xb, w_ref[...], preferred_element_type=jnp.float32)
    o_ref[0, 0, :] = res[:, 0] + b_ref[0, 0]


def _scores(x, w_sample, b_sample):
    xr = x.reshape(NROW, FSEG, D)
    br = b_sample.reshape(1, 1)
    return pl.pallas_call(
        _scores_body,
        grid=(NROW,),
        in_specs=[
            pl.BlockSpec((1, FSEG, D), lambda i: (i, 0, 0)),
            pl.BlockSpec((D, 1), lambda i: (0, 0)),
            pl.BlockSpec((1, 1), lambda i: (0, 0)),
        ],
        out_specs=pl.BlockSpec((1, 1, FSEG), lambda i: (i, 0, 0)),
        out_shape=jax.ShapeDtypeStruct((NROW, 1, FSEG), jnp.float32),
    )(xr, w_sample, br)


def _route_body(scores_hbm, sel_hbm, mask_hbm, sc_v, keys_v, sel_v, mask_v):
    wid = lax.axis_index("c") * 16 + lax.axis_index("s")

    @pl.when(wid < NROW)
    def _():
        row = wid
        pltpu.sync_copy(scores_hbm.at[row], sc_v)

        def key_chunk(i, _):
            s = sc_v[pl.ds(i * 16, 16)]
            b = lax.bitcast_convert_type(s, jnp.int32)
            keys_v[pl.ds(i * 16, 16)] = b ^ (
                (b >> jnp.int32(31)) & jnp.int32(0x7FFFFFFF))
            return 0

        lax.fori_loop(0, FSEG // 16, key_chunk, 0)

        zero16 = jnp.zeros((16,), jnp.int32)

        def count_ge(thr):
            def cc(i, acc):
                k = keys_v[pl.ds(i * 16, 16)]
                return acc + plsc.all_reduce_population_count(k >= thr)
            return lax.fori_loop(0, FSEG // 16, cc, zero16)

        def bit_step(j, t):
            cand = t + (jnp.int32(1) << (jnp.int32(31) - j))
            return jnp.where(count_ge(cand) >= SEG, cand, t)

        thr = lax.fori_loop(0, 32, bit_step,
                            jnp.full((16,), -2147483648, jnp.int32))

        def cg(i, acc):
            k = keys_v[pl.ds(i * 16, 16)]
            return acc + plsc.all_reduce_population_count(k > thr)

        n_gt = lax.fori_loop(0, FSEG // 16, cg, zero16)
        tneed = SEG - n_gt

        gbase = row * FSEG

        def comp(i, carry):
            nsel, neq = carry
            k = keys_v[pl.ds(i * 16, 16)]
            m_gt = k > thr
            m_eq = k == thr
            eqi = jnp.where(m_eq, jnp.int32(1), jnp.int32(0))
            eq_excl = plsc.cumsum(eqi) - eqi
            take_eq = jnp.logical_and(m_eq, (neq + eq_excl) < tneed)
            keep = jnp.logical_or(m_gt, take_eq)
            ki = jnp.where(keep, jnp.int32(1), jnp.int32(0))
            pos = nsel + (plsc.cumsum(ki) - ki)
            gidx = gbase + i * 16 + lax.iota(jnp.int32, 16)
            plsc.store_scatter(sel_v, [pos], gidx, mask=keep)
            mask_v[pl.ds(i * 16, 16)] = jnp.where(keep, 1.0, 0.0)
            return (nsel + plsc.all_reduce_population_count(keep),
                    neq + plsc.all_reduce_population_count(m_eq))

        lax.fori_loop(0, FSEG // 16, comp, (zero16, zero16))
        pltpu.sync_copy(sel_v, sel_hbm.at[row])
        pltpu.sync_copy(mask_v, mask_hbm.at[row])


def _route(scores):
    mesh = plsc.VectorSubcoreMesh(core_axis_name="c", subcore_axis_name="s")
    f = pl.kernel(
        _route_body,
        out_type=(
            jax.ShapeDtypeStruct((NROW, SEG), jnp.int32),
            jax.ShapeDtypeStruct((NROW, FSEG), jnp.float32),
        ),
        mesh=mesh,
        compiler_params=pltpu.CompilerParams(needs_layout_passes=False),
        scratch_types=[
            pltpu.VMEM((FSEG,), jnp.float32),
            pltpu.VMEM((FSEG,), jnp.int32),
            pltpu.VMEM((SEG,), jnp.int32),
            pltpu.VMEM((FSEG,), jnp.float32),
        ],
    )
    return f(scores)


_GROWS = TSEL // 32


def _gather_body(xflat_hbm, selg_hbm, out_hbm, idx_v, rows_v, sem):
    wid = lax.axis_index("c") * 16 + lax.axis_index("s")
    base = wid * _GROWS
    pltpu.sync_copy(selg_hbm.at[pl.ds(base, _GROWS)], idx_v)
    pltpu.async_copy(xflat_hbm.at[idx_v], rows_v, sem).wait()
    pltpu.sync_copy(rows_v, out_hbm.at[pl.ds(base, _GROWS)])


def _gather(xflat, selg):
    mesh = plsc.VectorSubcoreMesh(core_axis_name="c", subcore_axis_name="s")
    f = pl.kernel(
        _gather_body,
        out_type=jax.ShapeDtypeStruct((TSEL, D), jnp.float32),
        mesh=mesh,
        scratch_types=[
            pltpu.VMEM((_GROWS,), jnp.int32),
            pltpu.VMEM((_GROWS, D), jnp.float32),
            pltpu.SemaphoreType.DMA,
        ],
    )
    return f(xflat, selg)


def _qkv_body(x_ref, w_ref, o_ref):
    o_ref[...] = jnp.dot(x_ref[...].astype(jnp.bfloat16), w_ref[...],
                         preferred_element_type=jnp.float32)


def _qkv(x_sel, wqkv):
    return pl.pallas_call(
        _qkv_body,
        grid=(TSEL // SEG,),
        in_specs=[
            pl.BlockSpec((SEG, D), lambda i: (i, 0)),
            pl.BlockSpec((D, 3 * H * DK), lambda i: (0, 0)),
        ],
        out_specs=pl.BlockSpec((SEG, 3 * H * DK), lambda i: (i, 0)),
        out_shape=jax.ShapeDtypeStruct((TSEL, 3 * H * DK), jnp.float32),
    )(x_sel, wqkv.astype(jnp.bfloat16))


def _attn_body(qkv_ref, beta_ref, o_ref):
    beta = jax.nn.sigmoid(beta_ref[...])
    n_seg = NSEL // SEG
    for h in range(H):
        q = qkv_ref[0, :, h * DK:(h + 1) * DK]
        k = qkv_ref[0, :, H * DK + h * DK: H * DK + (h + 1) * DK]
        v = qkv_ref[0, :, 2 * H * DK + h * DK: 2 * H * DK + (h + 1) * DK]
        bh = beta[h:h + 1, :]
        mem = jnp.zeros((DK, DV), jnp.float32)
        zrow = jnp.full((1, DK), 1.0 / DK, jnp.float32)
        for i in range(n_seg):
            qs = q[i * SEG:(i + 1) * SEG, :]
            ks = k[i * SEG:(i + 1) * SEG, :]
            vs = v[i * SEG:(i + 1) * SEG, :]
            sq = jnp.where(qs > 0, qs + 1.0, jnp.exp(qs))
            sk = jnp.where(ks > 0, ks + 1.0, jnp.exp(ks))
            sc = lax.dot_general(qs, ks, (((1,), (1,)), ((), ())),
                                 preferred_element_type=jnp.float32)
            sc = sc * (1.0 / (DK ** 0.5))
            ri = lax.broadcasted_iota(jnp.int32, (SEG, SEG), 0)
            ci = lax.broadcasted_iota(jnp.int32, (SEG, SEG), 1)
            sc = jnp.where(ri >= ci, sc, -jnp.inf)
            m = jnp.max(sc, axis=1, keepdims=True)
            e = jnp.exp(sc - m)
            p = e / jnp.sum(e, axis=1, keepdims=True)
            att_dot = jnp.dot(p, vs, preferred_element_type=jnp.float32)
            num = jnp.dot(sq, mem, preferred_element_type=jnp.float32)
            den = jnp.sum(sq * zrow, axis=1, keepdims=True)
            att_mem = num / den
            mem = mem + lax.dot_general(sk, vs, (((0,), (0,)), ((), ())),
                                        preferred_element_type=jnp.float32)
            zrow = zrow + jnp.sum(sk, axis=0, keepdims=True)
            att = bh * att_mem + (1.0 - bh) * att_dot
            o_ref[0, h, i * SEG:(i + 1) * SEG, :] = att


def _attn(qkv, betas):
    qkvr = qkv.reshape(B, NSEL, 3 * H * DK)
    betar = betas.reshape(H, DV)
    return pl.pallas_call(
        _attn_body,
        grid=(B,),
        in_specs=[
            pl.BlockSpec((1, NSEL, 3 * H * DK), lambda b: (b, 0, 0)),
            pl.BlockSpec((H, DV), lambda b: (0, 0)),
        ],
        out_specs=pl.BlockSpec((1, H, NSEL, DV), lambda b: (b, 0, 0, 0)),
        out_shape=jax.ShapeDtypeStruct((B, H, NSEL, DV), jnp.float32),
    )(qkvr, betar)


def _proj_mlp_body(att_ref, wo_ref, w1_ref, b1_ref, w2_ref, b2_ref, o_ref):
    acc = jnp.zeros((SEG, D), jnp.float32)
    for h in range(H):
        acc = acc + jnp.dot(att_ref[0, h].astype(jnp.bfloat16),
                            wo_ref[h * DV:(h + 1) * DV, :],
                            preferred_element_type=jnp.float32)
    g = jax.nn.gelu(jnp.dot(acc.astype(jnp.bfloat16), w1_ref[...],
                            preferred_element_type=jnp.float32) + b1_ref[...])
    o_ref[0] = jnp.dot(g.astype(jnp.bfloat16), w2_ref[...],
                       preferred_element_type=jnp.float32) + b2_ref[...]


def _proj_mlp(att, wo, w1, b1, w2, b2):
    n_t = NSEL // SEG
    return pl.pallas_call(
        _proj_mlp_body,
        grid=(B, n_t),
        in_specs=[
            pl.BlockSpec((1, H, SEG, DV), lambda b, t: (b, 0, t, 0)),
            pl.BlockSpec((H * DV, D), lambda b, t: (0, 0)),
            pl.BlockSpec((D, HID), lambda b, t: (0, 0)),
            pl.BlockSpec((1, HID), lambda b, t: (0, 0)),
            pl.BlockSpec((HID, D), lambda b, t: (0, 0)),
            pl.BlockSpec((1, D), lambda b, t: (0, 0)),
        ],
        out_specs=pl.BlockSpec((1, SEG, D), lambda b, t: (b * n_t + t, 0, 0)),
        out_shape=jax.ShapeDtypeStruct((B * n_t, SEG, D), jnp.float32),
    )(att, wo.astype(jnp.bfloat16), w1.reshape(D, HID).astype(jnp.bfloat16),
      b1.reshape(1, HID), w2.reshape(HID, D).astype(jnp.bfloat16),
      b2.reshape(1, D))


def _scatter_ln_body(x_ref, h_ref, sel_ref, lnw_ref, lnb_ref, o_ref):
    r = pl.program_id(0)
    sel_loc = sel_ref[0, 0, :] - r * FSEG
    rows = lax.broadcasted_iota(jnp.int32, (FSEG, SEG), 0)
    oh = (rows == sel_loc[None, :]).astype(jnp.float32)
    scat = jnp.dot(oh, h_ref[0], preferred_element_type=jnp.float32)
    y = x_ref[0] + scat
    mu = jnp.mean(y, axis=1, keepdims=True)
    d = y - mu
    var = jnp.mean(d * d, axis=1, keepdims=True)
    o_ref[0] = d * lax.rsqrt(var + 1e-5) * lnw_ref[...] + lnb_ref[...]


def _scatter_ln(x, hsel, sel, ln_w, ln_b):
    xr = x.reshape(NROW, FSEG, D)
    hr = hsel.reshape(NROW, SEG, D)
    selr = sel.reshape(NROW, 1, SEG)
    return pl.pallas_call(
        _scatter_ln_body,
        grid=(NROW,),
        in_specs=[
            pl.BlockSpec((1, FSEG, D), lambda i: (i, 0, 0)),
            pl.BlockSpec((1, SEG, D), lambda i: (i, 0, 0)),
            pl.BlockSpec((1, 1, SEG), lambda i: (i, 0, 0)),
            pl.BlockSpec((1, D), lambda i: (0, 0)),
            pl.BlockSpec((1, D), lambda i: (0, 0)),
        ],
        out_specs=pl.BlockSpec((1, FSEG, D), lambda i: (i, 0, 0)),
        out_shape=jax.ShapeDtypeStruct((NROW, FSEG, D), jnp.float32),
    )(xr, hr, selr, ln_w.reshape(1, D), ln_b.reshape(1, D))


def kernel(x, W_sample, b_sample, Wq, Wk, Wv, Wo, betas, W1, b1, W2, b2,
           ln_w, ln_b):
    scores8 = _scores(x, W_sample, b_sample)
    sel, mask8 = _route(scores8.reshape(NROW, FSEG))
    x_sel = _gather(x.reshape(B * S, D), sel.reshape(TSEL))
    wqkv = jnp.concatenate([Wq, Wk, Wv], axis=1)
    qkv = _qkv(x_sel, wqkv)
    att = _attn(qkv, betas)
    hsel = _proj_mlp(att, Wo, W1, b1, W2, b2)
    out = _scatter_ln(x, hsel, sel, ln_w, ln_b)
    return (out.reshape(B, S, D),
            mask8.reshape(B * S, 1),
            scores8.reshape(B * S, 1))


# speedup vs baseline: 1.2970x; 1.2970x over previous
import functools

import jax
import jax.numpy as jnp
from jax import lax
from jax.experimental import pallas as pl
from jax.experimental.pallas import tpu as pltpu
from jax.experimental.pallas import tpu_sc as plsc

D = 1024
HID = 2048
DK = 64
DV = 64
H = 16
FSEG = 2048
SEG = 256
B = 2
S = 8192
NROW = B * (S // FSEG)
NSEL = S // 8
TSEL = B * NSEL


def _scores_body(x_ref, w_ref, b_ref, o_ref):
    xb = x_ref[0]
    res = jnp.dot(xb, w_ref[...], preferred_element_type=jnp.float32)
    o_ref[0, 0, :] = res[:, 0] + b_ref[0, 0]


def _scores(x, w_sample, b_sample):
    xr = x.reshape(NROW, FSEG, D)
    br = b_sample.reshape(1, 1)
    return pl.pallas_call(
        _scores_body,
        grid=(NROW,),
        in_specs=[
            pl.BlockSpec((1, FSEG, D), lambda i: (i, 0, 0)),
            pl.BlockSpec((D, 1), lambda i: (0, 0)),
            pl.BlockSpec((1, 1), lambda i: (0, 0)),
        ],
        out_specs=pl.BlockSpec((1, 1, FSEG), lambda i: (i, 0, 0)),
        out_shape=jax.ShapeDtypeStruct((NROW, 1, FSEG), jnp.float32),
    )(xr, w_sample, br)


def _route_body(scores_hbm, sel_hbm, mask_hbm, sc_v, keys_v, sel_v, mask_v):
    wid = lax.axis_index("c") * 16 + lax.axis_index("s")

    @pl.when(wid < NROW)
    def _():
        row = wid
        pltpu.sync_copy(scores_hbm.at[row], sc_v)

        def key_chunk(i, _):
            s = sc_v[pl.ds(i * 16, 16)]
            b = lax.bitcast_convert_type(s, jnp.int32)
            keys_v[pl.ds(i * 16, 16)] = b ^ (
                (b >> jnp.int32(31)) & jnp.int32(0x7FFFFFFF))
            return 0

        lax.fori_loop(0, FSEG // 16, key_chunk, 0)

        zero16 = jnp.zeros((16,), jnp.int32)

        def count_ge(thr):
            def cc(i, acc):
                k = keys_v[pl.ds(i * 16, 16)]
                return acc + plsc.all_reduce_population_count(k >= thr)
            return lax.fori_loop(0, FSEG // 16, cc, zero16)

        def bit_step(j, t):
            cand = t + (jnp.int32(1) << (jnp.int32(31) - j))
            return jnp.where(count_ge(cand) >= SEG, cand, t)

        thr = lax.fori_loop(0, 32, bit_step,
                            jnp.full((16,), -2147483648, jnp.int32))

        def cg(i, acc):
            k = keys_v[pl.ds(i * 16, 16)]
            return acc + plsc.all_reduce_population_count(k > thr)

        n_gt = lax.fori_loop(0, FSEG // 16, cg, zero16)
        tneed = SEG - n_gt

        gbase = row * FSEG

        def comp(i, carry):
            nsel, neq = carry
            k = keys_v[pl.ds(i * 16, 16)]
            m_gt = k > thr
            m_eq = k == thr
            eqi = jnp.where(m_eq, jnp.int32(1), jnp.int32(0))
            eq_excl = plsc.cumsum(eqi) - eqi
            take_eq = jnp.logical_and(m_eq, (neq + eq_excl) < tneed)
            keep = jnp.logical_or(m_gt, take_eq)
            ki = jnp.where(keep, jnp.int32(1), jnp.int32(0))
            pos = nsel + (plsc.cumsum(ki) - ki)
            gidx = gbase + i * 16 + lax.iota(jnp.int32, 16)
            plsc.store_scatter(sel_v, [pos], gidx, mask=keep)
            mask_v[pl.ds(i * 16, 16)] = jnp.where(keep, 1.0, 0.0)
            return (nsel + plsc.all_reduce_population_count(keep),
                    neq + plsc.all_reduce_population_count(m_eq))

        lax.fori_loop(0, FSEG // 16, comp, (zero16, zero16))
        pltpu.sync_copy(sel_v, sel_hbm.at[row])
        pltpu.sync_copy(mask_v, mask_hbm.at[row])


def _route(scores):
    mesh = plsc.VectorSubcoreMesh(core_axis_name="c", subcore_axis_name="s")
    f = pl.kernel(
        _route_body,
        out_type=(
            jax.ShapeDtypeStruct((NROW, SEG), jnp.int32),
            jax.ShapeDtypeStruct((NROW, FSEG), jnp.float32),
        ),
        mesh=mesh,
        compiler_params=pltpu.CompilerParams(needs_layout_passes=False),
        scratch_types=[
            pltpu.VMEM((FSEG,), jnp.float32),
            pltpu.VMEM((FSEG,), jnp.int32),
            pltpu.VMEM((SEG,), jnp.int32),
            pltpu.VMEM((FSEG,), jnp.float32),
        ],
    )
    return f(scores)


_GROWS = TSEL // 32


def _gather_body(xflat_hbm, selg_hbm, out_hbm, idx_v, rows_v, sem):
    wid = lax.axis_index("c") * 16 + lax.axis_index("s")
    base = wid * _GROWS
    pltpu.sync_copy(selg_hbm.at[pl.ds(base, _GROWS)], idx_v)
    pltpu.async_copy(xflat_hbm.at[idx_v], rows_v, sem).wait()
    pltpu.sync_copy(rows_v, out_hbm.at[pl.ds(base, _GROWS)])


def _gather(xflat, selg):
    mesh = plsc.VectorSubcoreMesh(core_axis_name="c", subcore_axis_name="s")
    f = pl.kernel(
        _gather_body,
        out_type=jax.ShapeDtypeStruct((TSEL, D), jnp.float32),
        mesh=mesh,
        scratch_types=[
            pltpu.VMEM((_GROWS,), jnp.int32),
            pltpu.VMEM((_GROWS, D), jnp.float32),
            pltpu.SemaphoreType.DMA,
        ],
    )
    return f(xflat, selg)


def _attn_body(x_ref, w_ref, beta_ref, o_ref):
    qkv = jnp.dot(x_ref[0].astype(jnp.bfloat16), w_ref[...],
                  preferred_element_type=jnp.float32)
    beta = jax.nn.sigmoid(beta_ref[...])
    n_seg = NSEL // SEG
    ri = lax.broadcasted_iota(jnp.int32, (SEG, SEG), 0)
    ci = lax.broadcasted_iota(jnp.int32, (SEG, SEG), 1)
    causal = ri >= ci
    for h in range(H):
        q = qkv[:, h * DK:(h + 1) * DK]
        k = qkv[:, H * DK + h * DK: H * DK + (h + 1) * DK]
        v = qkv[:, 2 * H * DK + h * DK: 2 * H * DK + (h + 1) * DK]
        bh = beta[h:h + 1, :]
        mem = jnp.zeros((DK, DV), jnp.float32)
        zrow = jnp.full((1, DK), 1.0 / DK, jnp.float32)
        for i in range(n_seg):
            qs = q[i * SEG:(i + 1) * SEG, :]
            ks = k[i * SEG:(i + 1) * SEG, :]
            vs = v[i * SEG:(i + 1) * SEG, :]
            sq = jnp.where(qs > 0, qs + 1.0, jnp.exp(qs))
            sk = jnp.where(ks > 0, ks + 1.0, jnp.exp(ks))
            sc = lax.dot_general(qs, ks, (((1,), (1,)), ((), ())),
                                 preferred_element_type=jnp.float32)
            sc = sc * (1.0 / (DK ** 0.5))
            e = jnp.where(causal, jnp.exp(sc), 0.0)
            att_dot = jnp.dot(e, vs, preferred_element_type=jnp.float32)
            att_dot = att_dot / jnp.sum(e, axis=1, keepdims=True)
            num = jnp.dot(sq, mem, preferred_element_type=jnp.float32)
            den = jnp.sum(sq * zrow, axis=1, keepdims=True)
            att_mem = num / den
            mem = mem + lax.dot_general(sk, vs, (((0,), (0,)), ((), ())),
                                        preferred_element_type=jnp.float32)
            zrow = zrow + jnp.sum(sk, axis=0, keepdims=True)
            att = bh * att_mem + (1.0 - bh) * att_dot
            o_ref[0, i * SEG:(i + 1) * SEG, h * DV:(h + 1) * DV] = att


def _attn(x_sel, wqkv, betas):
    xr = x_sel.reshape(B, NSEL, D)
    betar = betas.reshape(H, DV)
    return pl.pallas_call(
        _attn_body,
        grid=(B,),
        in_specs=[
            pl.BlockSpec((1, NSEL, D), lambda b: (b, 0, 0)),
            pl.BlockSpec((D, 3 * H * DK), lambda b: (0, 0)),
            pl.BlockSpec((H, DV), lambda b: (0, 0)),
        ],
        out_specs=pl.BlockSpec((1, NSEL, H * DV), lambda b: (b, 0, 0)),
        out_shape=jax.ShapeDtypeStruct((B, NSEL, H * DV), jnp.float32),
    )(xr, wqkv.astype(jnp.bfloat16), betar)


def _tail_body(att_ref, x_ref, sel_ref, wo_ref, w1_ref, b1_ref, w2_ref,
               b2_ref, lnw_ref, lnb_ref, o_ref):
    t = jnp.dot(att_ref[0].astype(jnp.bfloat16), wo_ref[...],
                preferred_element_type=jnp.float32)
    g = jax.nn.gelu(jnp.dot(t.astype(jnp.bfloat16), w1_ref[...],
                            preferred_element_type=jnp.float32) + b1_ref[...])
    hh = jnp.dot(g.astype(jnp.bfloat16), w2_ref[...],
                 preferred_element_type=jnp.float32) + b2_ref[...]
    r = pl.program_id(0)
    sel_loc = sel_ref[0, 0, :] - r * FSEG
    rows = lax.broadcasted_iota(jnp.int32, (FSEG, SEG), 0)
    oh = (rows == sel_loc[None, :]).astype(jnp.float32)
    scat = jnp.dot(oh, hh, preferred_element_type=jnp.float32)
    y = x_ref[0] + scat
    mu = jnp.mean(y, axis=1, keepdims=True)
    d = y - mu
    var = jnp.mean(d * d, axis=1, keepdims=True)
    o_ref[0] = d * lax.rsqrt(var + 1e-5) * lnw_ref[...] + lnb_ref[...]


def _tail(att, x, sel, wo, w1, b1, w2, b2, ln_w, ln_b):
    attr = att.reshape(NROW, SEG, H * DV)
    xr = x.reshape(NROW, FSEG, D)
    selr = sel.reshape(NROW, 1, SEG)
    return pl.pallas_call(
        _tail_body,
        grid=(NROW,),
        in_specs=[
            pl.BlockSpec((1, SEG, H * DV), lambda i: (i, 0, 0)),
            pl.BlockSpec((1, FSEG, D), lambda i: (i, 0, 0)),
            pl.BlockSpec((1, 1, SEG), lambda i: (i, 0, 0)),
            pl.BlockSpec((H * DV, D), lambda i: (0, 0)),
            pl.BlockSpec((D, HID), lambda i: (0, 0)),
            pl.BlockSpec((1, HID), lambda i: (0, 0)),
            pl.BlockSpec((HID, D), lambda i: (0, 0)),
            pl.BlockSpec((1, D), lambda i: (0, 0)),
            pl.BlockSpec((1, D), lambda i: (0, 0)),
            pl.BlockSpec((1, D), lambda i: (0, 0)),
        ],
        out_specs=pl.BlockSpec((1, FSEG, D), lambda i: (i, 0, 0)),
        out_shape=jax.ShapeDtypeStruct((NROW, FSEG, D), jnp.float32),
    )(attr, xr, selr, wo.astype(jnp.bfloat16),
      w1.reshape(D, HID).astype(jnp.bfloat16), b1.reshape(1, HID),
      w2.reshape(HID, D).astype(jnp.bfloat16), b2.reshape(1, D),
      ln_w.reshape(1, D), ln_b.reshape(1, D))


def kernel(x, W_sample, b_sample, Wq, Wk, Wv, Wo, betas, W1, b1, W2, b2,
           ln_w, ln_b):
    scores8 = _scores(x, W_sample, b_sample)
    sel, mask8 = _route(scores8.reshape(NROW, FSEG))
    x_sel = _gather(x.reshape(B * S, D), sel.reshape(TSEL))
    wqkv = jnp.concatenate([Wq, Wk, Wv], axis=1)
    att = _attn(x_sel, wqkv, betas)
    out = _tail(att, x, sel, Wo, W1, b1, W2, b2, ln_w, ln_b)
    return (out.reshape(B, S, D),
            mask8.reshape(B * S, 1),
            scores8.reshape(B * S, 1))


# speedup vs baseline: 1.4876x; 1.1470x over previous
import functools

import jax
import jax.numpy as jnp
from jax import lax
from jax.experimental import pallas as pl
from jax.experimental.pallas import tpu as pltpu
from jax.experimental.pallas import tpu_sc as plsc

D = 1024
HID = 2048
DK = 64
DV = 64
H = 16
FSEG = 2048
SEG = 256
B = 2
S = 8192
NROW = B * (S // FSEG)
NSEL = S // 8
TSEL = B * NSEL


def _scores_body(x_ref, w_ref, b_ref, o_ref):
    xb = x_ref[0]
    res = jnp.dot(xb, w_ref[...], preferred_element_type=jnp.float32)
    o_ref[0, 0, :] = res[:, 0] + b_ref[0, 0]


def _scores(x, w_sample, b_sample):
    xr = x.reshape(NROW, FSEG, D)
    br = b_sample.reshape(1, 1)
    return pl.pallas_call(
        _scores_body,
        grid=(NROW,),
        in_specs=[
            pl.BlockSpec((1, FSEG, D), lambda i: (i, 0, 0)),
            pl.BlockSpec((D, 1), lambda i: (0, 0)),
            pl.BlockSpec((1, 1), lambda i: (0, 0)),
        ],
        out_specs=pl.BlockSpec((1, 1, FSEG), lambda i: (i, 0, 0)),
        out_shape=jax.ShapeDtypeStruct((NROW, 1, FSEG), jnp.float32),
    )(xr, w_sample, br)


def _route_body(scores_hbm, sel_hbm, mask_hbm, sc_v, keys_v, sel_v, mask_v):
    wid = lax.axis_index("c") * 16 + lax.axis_index("s")

    @pl.when(wid < NROW)
    def _():
        row = wid
        pltpu.sync_copy(scores_hbm.at[row], sc_v)

        UNROLL = 8

        def key_chunk(i, _):
            for u in range(UNROLL):
                s = sc_v[pl.ds((i * UNROLL + u) * 16, 16)]
                b = lax.bitcast_convert_type(s, jnp.int32)
                keys_v[pl.ds((i * UNROLL + u) * 16, 16)] = b ^ (
                    (b >> jnp.int32(31)) & jnp.int32(0x7FFFFFFF))
            return 0

        lax.fori_loop(0, FSEG // 16 // UNROLL, key_chunk, 0)

        zero16 = jnp.zeros((16,), jnp.int32)

        def count_ge(thr):
            def cc(i, acc):
                for u in range(UNROLL):
                    k = keys_v[pl.ds((i * UNROLL + u) * 16, 16)]
                    acc = acc + plsc.all_reduce_population_count(k >= thr)
                return acc
            return lax.fori_loop(0, FSEG // 16 // UNROLL, cc, zero16)

        def bit_step(j, t):
            cand = t + (jnp.int32(1) << (jnp.int32(31) - j))
            return jnp.where(count_ge(cand) >= SEG, cand, t)

        thr = lax.fori_loop(0, 32, bit_step,
                            jnp.full((16,), -2147483648, jnp.int32))

        def cg(i, acc):
            for u in range(UNROLL):
                k = keys_v[pl.ds((i * UNROLL + u) * 16, 16)]
                acc = acc + plsc.all_reduce_population_count(k > thr)
            return acc

        n_gt = lax.fori_loop(0, FSEG // 16 // UNROLL, cg, zero16)
        tneed = SEG - n_gt

        gbase = row * FSEG

        def comp(i, carry):
            nsel, neq = carry
            k = keys_v[pl.ds(i * 16, 16)]
            m_gt = k > thr
            m_eq = k == thr
            eqi = jnp.where(m_eq, jnp.int32(1), jnp.int32(0))
            eq_excl = plsc.cumsum(eqi) - eqi
            take_eq = jnp.logical_and(m_eq, (neq + eq_excl) < tneed)
            keep = jnp.logical_or(m_gt, take_eq)
            ki = jnp.where(keep, jnp.int32(1), jnp.int32(0))
            pos = nsel + (plsc.cumsum(ki) - ki)
            gidx = gbase + i * 16 + lax.iota(jnp.int32, 16)
            plsc.store_scatter(sel_v, [pos], gidx, mask=keep)
            mask_v[pl.ds(i * 16, 16)] = jnp.where(keep, 1.0, 0.0)
            return (nsel + plsc.all_reduce_population_count(keep),
                    neq + plsc.all_reduce_population_count(m_eq))

        lax.fori_loop(0, FSEG // 16, comp, (zero16, zero16))
        pltpu.sync_copy(sel_v, sel_hbm.at[row])
        pltpu.sync_copy(mask_v, mask_hbm.at[row])


def _route(scores):
    mesh = plsc.VectorSubcoreMesh(core_axis_name="c", subcore_axis_name="s")
    f = pl.kernel(
        _route_body,
        out_type=(
            jax.ShapeDtypeStruct((NROW, SEG), jnp.int32),
            jax.ShapeDtypeStruct((NROW, FSEG), jnp.float32),
        ),
        mesh=mesh,
        compiler_params=pltpu.CompilerParams(needs_layout_passes=False),
        scratch_types=[
            pltpu.VMEM((FSEG,), jnp.float32),
            pltpu.VMEM((FSEG,), jnp.int32),
            pltpu.VMEM((SEG,), jnp.int32),
            pltpu.VMEM((FSEG,), jnp.float32),
        ],
    )
    return f(scores)


_GROWS = TSEL // 32


def _gather_body(xflat_hbm, selg_hbm, out_hbm, idx_v, rows_v, sem):
    wid = lax.axis_index("c") * 16 + lax.axis_index("s")
    base = wid * _GROWS
    pltpu.sync_copy(selg_hbm.at[pl.ds(base, _GROWS)], idx_v)
    pltpu.async_copy(xflat_hbm.at[idx_v], rows_v, sem).wait()
    pltpu.sync_copy(rows_v, out_hbm.at[pl.ds(base, _GROWS)])


def _gather(xflat, selg):
    mesh = plsc.VectorSubcoreMesh(core_axis_name="c", subcore_axis_name="s")
    f = pl.kernel(
        _gather_body,
        out_type=jax.ShapeDtypeStruct((TSEL, D), jnp.float32),
        mesh=mesh,
        scratch_types=[
            pltpu.VMEM((_GROWS,), jnp.int32),
            pltpu.VMEM((_GROWS, D), jnp.float32),
            pltpu.SemaphoreType.DMA,
        ],
    )
    return f(xflat, selg)


def _attn_body(x_ref, w_ref, beta_ref, o_ref):
    qkv = jnp.dot(x_ref[0].astype(jnp.bfloat16), w_ref[...],
                  preferred_element_type=jnp.float32)
    beta = jax.nn.sigmoid(beta_ref[...])
    n_seg = NSEL // SEG
    ri = lax.broadcasted_iota(jnp.int32, (SEG, SEG), 0)
    ci = lax.broadcasted_iota(jnp.int32, (SEG, SEG), 1)
    causal = ri >= ci
    for h in range(H):
        q = qkv[:, h * DK:(h + 1) * DK]
        k = qkv[:, H * DK + h * DK: H * DK + (h + 1) * DK]
        v = qkv[:, 2 * H * DK + h * DK: 2 * H * DK + (h + 1) * DK]
        bh = beta[h:h + 1, :]
        mem = jnp.zeros((DK, DV), jnp.float32)
        zrow = jnp.full((1, DK), 1.0 / DK, jnp.float32)
        for i in range(n_seg):
            qs = q[i * SEG:(i + 1) * SEG, :]
            ks = k[i * SEG:(i + 1) * SEG, :]
            vs = v[i * SEG:(i + 1) * SEG, :].astype(jnp.bfloat16)
            sq = jnp.where(qs > 0, qs + 1.0, jnp.exp(qs))
            sk = jnp.where(ks > 0, ks + 1.0, jnp.exp(ks))
            sc = lax.dot_general(qs.astype(jnp.bfloat16),
                                 ks.astype(jnp.bfloat16),
                                 (((1,), (1,)), ((), ())),
                                 preferred_element_type=jnp.float32)
            sc = sc * (1.0 / (DK ** 0.5))
            e = jnp.where(causal, jnp.exp(sc), 0.0)
            att_dot = jnp.dot(e.astype(jnp.bfloat16), vs,
                              preferred_element_type=jnp.float32)
            att_dot = att_dot / jnp.sum(e, axis=1, keepdims=True)
            num = jnp.dot(sq.astype(jnp.bfloat16),
                          mem.astype(jnp.bfloat16),
                          preferred_element_type=jnp.float32)
            den = jnp.sum(sq * zrow, axis=1, keepdims=True)
            att_mem = num / den
            mem = mem + lax.dot_general(sk.astype(jnp.bfloat16), vs,
                                        (((0,), (0,)), ((), ())),
                                        preferred_element_type=jnp.float32)
            zrow = zrow + jnp.sum(sk, axis=0, keepdims=True)
            att = bh * att_mem + (1.0 - bh) * att_dot
            o_ref[0, i * SEG:(i + 1) * SEG, h * DV:(h + 1) * DV] = att


def _attn(x_sel, wqkv, betas):
    xr = x_sel.reshape(B, NSEL, D)
    betar = betas.reshape(H, DV)
    return pl.pallas_call(
        _attn_body,
        grid=(B,),
        in_specs=[
            pl.BlockSpec((1, NSEL, D), lambda b: (b, 0, 0)),
            pl.BlockSpec((D, 3 * H * DK), lambda b: (0, 0)),
            pl.BlockSpec((H, DV), lambda b: (0, 0)),
        ],
        out_specs=pl.BlockSpec((1, NSEL, H * DV), lambda b: (b, 0, 0)),
        out_shape=jax.ShapeDtypeStruct((B, NSEL, H * DV), jnp.float32),
    )(xr, wqkv.astype(jnp.bfloat16), betar)


def _tail_body(att_ref, x_ref, sel_ref, wo_ref, w1_ref, b1_ref, w2_ref,
               b2_ref, lnw_ref, lnb_ref, o_ref):
    t = jnp.dot(att_ref[0].astype(jnp.bfloat16), wo_ref[...],
                preferred_element_type=jnp.float32)
    g = jax.nn.gelu(jnp.dot(t.astype(jnp.bfloat16), w1_ref[...],
                            preferred_element_type=jnp.float32) + b1_ref[...])
    hh = jnp.dot(g.astype(jnp.bfloat16), w2_ref[...],
                 preferred_element_type=jnp.float32) + b2_ref[...]
    r = pl.program_id(0)
    sel_loc = sel_ref[0, 0, :] - r * FSEG
    rows = lax.broadcasted_iota(jnp.int32, (FSEG, SEG), 0)
    oh = (rows == sel_loc[None, :]).astype(jnp.float32)
    scat = jnp.dot(oh, hh, preferred_element_type=jnp.float32)
    y = x_ref[0] + scat
    mu = jnp.mean(y, axis=1, keepdims=True)
    d = y - mu
    var = jnp.mean(d * d, axis=1, keepdims=True)
    o_ref[0] = d * lax.rsqrt(var + 1e-5) * lnw_ref[...] + lnb_ref[...]


def _tail(att, x, sel, wo, w1, b1, w2, b2, ln_w, ln_b):
    attr = att.reshape(NROW, SEG, H * DV)
    xr = x.reshape(NROW, FSEG, D)
    selr = sel.reshape(NROW, 1, SEG)
    return pl.pallas_call(
        _tail_body,
        grid=(NROW,),
        in_specs=[
            pl.BlockSpec((1, SEG, H * DV), lambda i: (i, 0, 0)),
            pl.BlockSpec((1, FSEG, D), lambda i: (i, 0, 0)),
            pl.BlockSpec((1, 1, SEG), lambda i: (i, 0, 0)),
            pl.BlockSpec((H * DV, D), lambda i: (0, 0)),
            pl.BlockSpec((D, HID), lambda i: (0, 0)),
            pl.BlockSpec((1, HID), lambda i: (0, 0)),
            pl.BlockSpec((HID, D), lambda i: (0, 0)),
            pl.BlockSpec((1, D), lambda i: (0, 0)),
            pl.BlockSpec((1, D), lambda i: (0, 0)),
            pl.BlockSpec((1, D), lambda i: (0, 0)),
        ],
        out_specs=pl.BlockSpec((1, FSEG, D), lambda i: (i, 0, 0)),
        out_shape=jax.ShapeDtypeStruct((NROW, FSEG, D), jnp.float32),
    )(attr, xr, selr, wo.astype(jnp.bfloat16),
      w1.reshape(D, HID).astype(jnp.bfloat16), b1.reshape(1, HID),
      w2.reshape(HID, D).astype(jnp.bfloat16), b2.reshape(1, D),
      ln_w.reshape(1, D), ln_b.reshape(1, D))


def kernel(x, W_sample, b_sample, Wq, Wk, Wv, Wo, betas, W1, b1, W2, b2,
           ln_w, ln_b):
    scores8 = _scores(x, W_sample, b_sample)
    sel, mask8 = _route(scores8.reshape(NROW, FSEG))
    x_sel = _gather(x.reshape(B * S, D), sel.reshape(TSEL))
    wqkv = jnp.concatenate([Wq, Wk, Wv], axis=1)
    att = _attn(x_sel, wqkv, betas)
    out = _tail(att, x, sel, Wo, W1, b1, W2, b2, ln_w, ln_b)
    return (out.reshape(B, S, D),
            mask8.reshape(B * S, 1),
            scores8.reshape(B * S, 1))


# speedup vs baseline: 1.5295x; 1.0282x over previous
import functools

import jax
import jax.numpy as jnp
from jax import lax
from jax.experimental import pallas as pl
from jax.experimental.pallas import tpu as pltpu
from jax.experimental.pallas import tpu_sc as plsc

D = 1024
HID = 2048
DK = 64
DV = 64
H = 16
FSEG = 2048
SEG = 256
B = 2
S = 8192
NROW = B * (S // FSEG)
NSEL = S // 8
TSEL = B * NSEL


def _scores_body(x_ref, w_ref, b_ref, o_ref):
    xb = x_ref[0]
    res = jnp.dot(xb, w_ref[...], preferred_element_type=jnp.float32)
    o_ref[0, 0, :] = res[:, 0] + b_ref[0, 0]


def _scores(x, w_sample, b_sample):
    xr = x.reshape(NROW, FSEG, D)
    br = b_sample.reshape(1, 1)
    return pl.pallas_call(
        _scores_body,
        grid=(NROW,),
        in_specs=[
            pl.BlockSpec((1, FSEG, D), lambda i: (i, 0, 0)),
            pl.BlockSpec((D, 1), lambda i: (0, 0)),
            pl.BlockSpec((1, 1), lambda i: (0, 0)),
        ],
        out_specs=pl.BlockSpec((1, 1, FSEG), lambda i: (i, 0, 0)),
        out_shape=jax.ShapeDtypeStruct((NROW, 1, FSEG), jnp.float32),
    )(xr, w_sample, br)


def _route_body(scores_hbm, sel_hbm, mask_hbm, sc_v, keys_v, sel_v, mask_v):
    wid = lax.axis_index("c") * 16 + lax.axis_index("s")

    @pl.when(wid < NROW)
    def _():
        row = wid
        pltpu.sync_copy(scores_hbm.at[row], sc_v)

        UNROLL = 8

        def key_chunk(i, _):
            for u in range(UNROLL):
                s = sc_v[pl.ds((i * UNROLL + u) * 16, 16)]
                b = lax.bitcast_convert_type(s, jnp.int32)
                keys_v[pl.ds((i * UNROLL + u) * 16, 16)] = b ^ (
                    (b >> jnp.int32(31)) & jnp.int32(0x7FFFFFFF))
            return 0

        lax.fori_loop(0, FSEG // 16 // UNROLL, key_chunk, 0)

        zero16 = jnp.zeros((16,), jnp.int32)

        def count_ge(thr):
            def cc(i, acc):
                for u in range(UNROLL):
                    k = keys_v[pl.ds((i * UNROLL + u) * 16, 16)]
                    acc = acc + plsc.all_reduce_population_count(k >= thr)
                return acc
            return lax.fori_loop(0, FSEG // 16 // UNROLL, cc, zero16)

        def bit_step(j, t):
            cand = t + (jnp.int32(1) << (jnp.int32(31) - j))
            return jnp.where(count_ge(cand) >= SEG, cand, t)

        thr = lax.fori_loop(0, 32, bit_step,
                            jnp.full((16,), -2147483648, jnp.int32))

        def cg(i, acc):
            for u in range(UNROLL):
                k = keys_v[pl.ds((i * UNROLL + u) * 16, 16)]
                acc = acc + plsc.all_reduce_population_count(k > thr)
            return acc

        n_gt = lax.fori_loop(0, FSEG // 16 // UNROLL, cg, zero16)
        tneed = SEG - n_gt

        gbase = row * FSEG

        def comp(i, carry):
            nsel, neq = carry
            k = keys_v[pl.ds(i * 16, 16)]
            m_gt = k > thr
            m_eq = k == thr
            eqi = jnp.where(m_eq, jnp.int32(1), jnp.int32(0))
            eq_excl = plsc.cumsum(eqi) - eqi
            take_eq = jnp.logical_and(m_eq, (neq + eq_excl) < tneed)
            keep = jnp.logical_or(m_gt, take_eq)
            ki = jnp.where(keep, jnp.int32(1), jnp.int32(0))
            pos = nsel + (plsc.cumsum(ki) - ki)
            gidx = gbase + i * 16 + lax.iota(jnp.int32, 16)
            plsc.store_scatter(sel_v, [pos], gidx, mask=keep)
            mask_v[pl.ds(i * 16, 16)] = jnp.where(keep, 1.0, 0.0)
            return (nsel + plsc.all_reduce_population_count(keep),
                    neq + plsc.all_reduce_population_count(m_eq))

        lax.fori_loop(0, FSEG // 16, comp, (zero16, zero16))
        pltpu.sync_copy(sel_v, sel_hbm.at[row])
        pltpu.sync_copy(mask_v, mask_hbm.at[row])


def _route(scores):
    mesh = plsc.VectorSubcoreMesh(core_axis_name="c", subcore_axis_name="s")
    f = pl.kernel(
        _route_body,
        out_type=(
            jax.ShapeDtypeStruct((NROW, SEG), jnp.int32),
            jax.ShapeDtypeStruct((NROW, FSEG), jnp.float32),
        ),
        mesh=mesh,
        compiler_params=pltpu.CompilerParams(needs_layout_passes=False),
        scratch_types=[
            pltpu.VMEM((FSEG,), jnp.float32),
            pltpu.VMEM((FSEG,), jnp.int32),
            pltpu.VMEM((SEG,), jnp.int32),
            pltpu.VMEM((FSEG,), jnp.float32),
        ],
    )
    return f(scores)


_GROWS = TSEL // 32


def _gather_body(xflat_hbm, selg_hbm, out_hbm, idx_v, rows_v, sem):
    wid = lax.axis_index("c") * 16 + lax.axis_index("s")
    base = wid * _GROWS
    pltpu.sync_copy(selg_hbm.at[pl.ds(base, _GROWS)], idx_v)
    pltpu.async_copy(xflat_hbm.at[idx_v], rows_v, sem).wait()
    pltpu.sync_copy(rows_v, out_hbm.at[pl.ds(base, _GROWS)])


def _gather(xflat, selg):
    mesh = plsc.VectorSubcoreMesh(core_axis_name="c", subcore_axis_name="s")
    f = pl.kernel(
        _gather_body,
        out_type=jax.ShapeDtypeStruct((TSEL, D), jnp.float32),
        mesh=mesh,
        scratch_types=[
            pltpu.VMEM((_GROWS,), jnp.int32),
            pltpu.VMEM((_GROWS, D), jnp.float32),
            pltpu.SemaphoreType.DMA,
        ],
    )
    return f(xflat, selg)


def _attn_body(x_ref, w_ref, beta_ref, o_ref):
    qkv = jnp.dot(x_ref[0].astype(jnp.bfloat16), w_ref[...],
                  preferred_element_type=jnp.float32)
    beta = jax.nn.sigmoid(beta_ref[...])
    n_seg = NSEL // SEG
    ri = lax.broadcasted_iota(jnp.int32, (SEG, SEG), 0)
    ci = lax.broadcasted_iota(jnp.int32, (SEG, SEG), 1)
    causal = ri >= ci
    for h in range(H):
        q = qkv[:, h * DK:(h + 1) * DK]
        k = qkv[:, H * DK + h * DK: H * DK + (h + 1) * DK]
        v = qkv[:, 2 * H * DK + h * DK: 2 * H * DK + (h + 1) * DK]
        bh = beta[h:h + 1, :]
        mem_aug = jnp.concatenate(
            [jnp.zeros((DK, DV), jnp.float32),
             jnp.full((DK, 1), 1.0 / DK, jnp.float32)], axis=1)
        ones_col = jnp.ones((SEG, 1), jnp.bfloat16)
        for i in range(n_seg):
            qs = q[i * SEG:(i + 1) * SEG, :]
            ks = k[i * SEG:(i + 1) * SEG, :]
            vs = v[i * SEG:(i + 1) * SEG, :].astype(jnp.bfloat16)
            vs_aug = jnp.concatenate([vs, ones_col], axis=1)
            sq = jnp.where(qs > 0, qs + 1.0, jnp.exp(qs))
            sk = jnp.where(ks > 0, ks + 1.0, jnp.exp(ks))
            sc = lax.dot_general(
                (qs * (1.0 / (DK ** 0.5))).astype(jnp.bfloat16),
                ks.astype(jnp.bfloat16),
                (((1,), (1,)), ((), ())),
                preferred_element_type=jnp.float32)
            e = jnp.where(causal, jnp.exp(sc), 0.0)
            da = jnp.dot(e.astype(jnp.bfloat16), vs_aug,
                         preferred_element_type=jnp.float32)
            att_dot = da[:, :DV] / da[:, DV:DV + 1]
            nd = jnp.dot(sq.astype(jnp.bfloat16),
                         mem_aug.astype(jnp.bfloat16),
                         preferred_element_type=jnp.float32)
            att_mem = nd[:, :DV] / nd[:, DV:DV + 1]
            mem_aug = mem_aug + lax.dot_general(
                sk.astype(jnp.bfloat16), vs_aug,
                (((0,), (0,)), ((), ())),
                preferred_element_type=jnp.float32)
            att = bh * att_mem + (1.0 - bh) * att_dot
            o_ref[0, i * SEG:(i + 1) * SEG, h * DV:(h + 1) * DV] = att


def _attn(x_sel, wqkv, betas):
    xr = x_sel.reshape(B, NSEL, D)
    betar = betas.reshape(H, DV)
    return pl.pallas_call(
        _attn_body,
        grid=(B,),
        in_specs=[
            pl.BlockSpec((1, NSEL, D), lambda b: (b, 0, 0)),
            pl.BlockSpec((D, 3 * H * DK), lambda b: (0, 0)),
            pl.BlockSpec((H, DV), lambda b: (0, 0)),
        ],
        out_specs=pl.BlockSpec((1, NSEL, H * DV), lambda b: (b, 0, 0)),
        out_shape=jax.ShapeDtypeStruct((B, NSEL, H * DV), jnp.float32),
    )(xr, wqkv.astype(jnp.bfloat16), betar)


def _tail_body(att_ref, x_ref, sel_ref, wo_ref, w1_ref, b1_ref, w2_ref,
               b2_ref, lnw_ref, lnb_ref, o_ref):
    t = jnp.dot(att_ref[0].astype(jnp.bfloat16), wo_ref[...],
                preferred_element_type=jnp.float32)
    g = jax.nn.gelu(jnp.dot(t.astype(jnp.bfloat16), w1_ref[...],
                            preferred_element_type=jnp.float32) + b1_ref[...])
    hh = jnp.dot(g.astype(jnp.bfloat16), w2_ref[...],
                 preferred_element_type=jnp.float32) + b2_ref[...]
    r = pl.program_id(0)
    sel_loc = sel_ref[0, 0, :] - r * FSEG
    rows = lax.broadcasted_iota(jnp.int32, (FSEG, SEG), 0)
    oh = (rows == sel_loc[None, :]).astype(jnp.float32)
    scat = jnp.dot(oh, hh, preferred_element_type=jnp.float32)
    y = x_ref[0] + scat
    mu = jnp.mean(y, axis=1, keepdims=True)
    d = y - mu
    var = jnp.mean(d * d, axis=1, keepdims=True)
    o_ref[0] = d * lax.rsqrt(var + 1e-5) * lnw_ref[...] + lnb_ref[...]


def _tail(att, x, sel, wo, w1, b1, w2, b2, ln_w, ln_b):
    attr = att.reshape(NROW, SEG, H * DV)
    xr = x.reshape(NROW, FSEG, D)
    selr = sel.reshape(NROW, 1, SEG)
    return pl.pallas_call(
        _tail_body,
        grid=(NROW,),
        in_specs=[
            pl.BlockSpec((1, SEG, H * DV), lambda i: (i, 0, 0)),
            pl.BlockSpec((1, FSEG, D), lambda i: (i, 0, 0)),
            pl.BlockSpec((1, 1, SEG), lambda i: (i, 0, 0)),
            pl.BlockSpec((H * DV, D), lambda i: (0, 0)),
            pl.BlockSpec((D, HID), lambda i: (0, 0)),
            pl.BlockSpec((1, HID), lambda i: (0, 0)),
            pl.BlockSpec((HID, D), lambda i: (0, 0)),
            pl.BlockSpec((1, D), lambda i: (0, 0)),
            pl.BlockSpec((1, D), lambda i: (0, 0)),
            pl.BlockSpec((1, D), lambda i: (0, 0)),
        ],
        out_specs=pl.BlockSpec((1, FSEG, D), lambda i: (i, 0, 0)),
        out_shape=jax.ShapeDtypeStruct((NROW, FSEG, D), jnp.float32),
    )(attr, xr, selr, wo.astype(jnp.bfloat16),
      w1.reshape(D, HID).astype(jnp.bfloat16), b1.reshape(1, HID),
      w2.reshape(HID, D).astype(jnp.bfloat16), b2.reshape(1, D),
      ln_w.reshape(1, D), ln_b.reshape(1, D))


def kernel(x, W_sample, b_sample, Wq, Wk, Wv, Wo, betas, W1, b1, W2, b2,
           ln_w, ln_b):
    scores8 = _scores(x, W_sample, b_sample)
    sel, mask8 = _route(scores8.reshape(NROW, FSEG))
    x_sel = _gather(x.reshape(B * S, D), sel.reshape(TSEL))
    wqkv = jnp.concatenate([Wq, Wk, Wv], axis=1)
    att = _attn(x_sel, wqkv, betas)
    out = _tail(att, x, sel, Wo, W1, b1, W2, b2, ln_w, ln_b)
    return (out.reshape(B, S, D),
            mask8.reshape(B * S, 1),
            scores8.reshape(B * S, 1))
